# Initial kernel scaffold; baseline (speedup 1.0000x reference)
import jax
import jax.numpy as jnp
from jax.experimental import pallas as pl


def kernel(x, feat_before_pooling, feat_after_pooling):
    raise NotImplementedError("write your pallas kernel here")


# speedup vs baseline: 5.7577x; 5.7577x over previous
import functools

import jax
import jax.numpy as jnp
from jax.experimental import pallas as pl

_K = 20
_TEMP = 0.01


def _block_kernel(feat_ref, sfeat_ref, x_ref, out_ref):
    f = feat_ref[0]
    s = sfeat_ref[0]
    xb = x_ref[0]

    fn = jnp.sum(f * f, axis=1, keepdims=True)
    sn = jnp.sum(s * s, axis=1, keepdims=True).T
    dot = jax.lax.dot_general(
        f, s, (((1,), (1,)), ((), ())),
        preferred_element_type=jnp.float32)
    ds = jnp.maximum(fn + sn - 2.0 * dot, 0.0)
    w = jnp.exp(-_TEMP * ds)

    bn, m = w.shape
    iota = jax.lax.broadcasted_iota(jnp.int32, (bn, m), 1)
    wm = w
    for _ in range(_K - 1):
        mx = jnp.max(wm, axis=1, keepdims=True)
        eq = wm == mx
        first = jnp.min(jnp.where(eq, iota, m), axis=1, keepdims=True)
        wm = jnp.where(iota == first, -jnp.inf, wm)
    thr = jnp.max(wm, axis=1, keepdims=True)

    att = jnp.where(w >= thr, w, 0.0)
    norm = jnp.sqrt(jnp.sum(att * att, axis=1, keepdims=True))
    att = att / jnp.maximum(norm, 1e-12)

    out_ref[0] = jax.lax.dot_general(
        att, xb, (((1,), (0,)), ((), ())),
        preferred_element_type=jnp.float32)


@functools.partial(jax.jit, static_argnames=("bn",))
def _run(x, feat, sfeat, bn):
    b, n, c = feat.shape
    _, m, _ = x.shape
    grid = (b, n // bn)
    return pl.pallas_call(
        _block_kernel,
        grid=grid,
        in_specs=[
            pl.BlockSpec((1, bn, c), lambda bi, ni: (bi, ni, 0)),
            pl.BlockSpec((1, m, c), lambda bi, ni: (bi, 0, 0)),
            pl.BlockSpec((1, m, c), lambda bi, ni: (bi, 0, 0)),
        ],
        out_specs=pl.BlockSpec((1, bn, c), lambda bi, ni: (bi, ni, 0)),
        out_shape=jax.ShapeDtypeStruct((b, n, c), jnp.float32),
    )(feat, sfeat, x)


def kernel(x, feat_before_pooling, feat_after_pooling):
    n = feat_before_pooling.shape[1]
    bn = 512 if n % 512 == 0 else n
    return _run(x, feat_before_pooling, feat_after_pooling, bn)


# speedup vs baseline: 7.4084x; 1.2867x over previous
import functools

import jax
import jax.numpy as jnp
from jax.experimental import pallas as pl

_K = 20
_TEMP = 0.01


def _block_kernel(feat_ref, sfeat_ref, x_ref, out_ref):
    f = feat_ref[0]
    s = sfeat_ref[0]
    xb = x_ref[0]

    fn = jnp.sum(f * f, axis=1, keepdims=True)
    sn = jnp.sum(s * s, axis=1, keepdims=True).T
    dot = jax.lax.dot_general(
        f, s, (((1,), (1,)), ((), ())),
        preferred_element_type=jnp.float32)
    ds = jnp.maximum(fn + sn - 2.0 * dot, 0.0)
    w = jnp.exp(-_TEMP * ds)

    bn, m = w.shape
    wm = w
    removed = jnp.zeros((bn, 1), jnp.float32)
    thr = jnp.zeros((bn, 1), jnp.float32)
    for j in range(_K):
        mx = jnp.max(wm, axis=1, keepdims=True)
        thr = jnp.where(removed < _K, mx, thr)
        if j < _K - 1:
            eq = wm == mx
            removed = removed + jnp.sum(
                jnp.where(eq, 1.0, 0.0), axis=1, keepdims=True)
            wm = jnp.where(eq, -jnp.inf, wm)

    att = jnp.where(w >= thr, w, 0.0)
    norm = jnp.sqrt(jnp.sum(att * att, axis=1, keepdims=True))
    att = att / jnp.maximum(norm, 1e-12)

    out_ref[0] = jax.lax.dot_general(
        att, xb, (((1,), (0,)), ((), ())),
        preferred_element_type=jnp.float32)


@functools.partial(jax.jit, static_argnames=("bn",))
def _run(x, feat, sfeat, bn):
    b, n, c = feat.shape
    _, m, _ = x.shape
    grid = (b, n // bn)
    return pl.pallas_call(
        _block_kernel,
        grid=grid,
        in_specs=[
            pl.BlockSpec((1, bn, c), lambda bi, ni: (bi, ni, 0)),
            pl.BlockSpec((1, m, c), lambda bi, ni: (bi, 0, 0)),
            pl.BlockSpec((1, m, c), lambda bi, ni: (bi, 0, 0)),
        ],
        out_specs=pl.BlockSpec((1, bn, c), lambda bi, ni: (bi, ni, 0)),
        out_shape=jax.ShapeDtypeStruct((b, n, c), jnp.float32),
    )(feat, sfeat, x)


def kernel(x, feat_before_pooling, feat_after_pooling):
    n = feat_before_pooling.shape[1]
    bn = 512 if n % 512 == 0 else n
    return _run(x, feat_before_pooling, feat_after_pooling, bn)
